# baseline (speedup 1.0000x reference)
import functools

import jax
import jax.numpy as jnp
import numpy as np
from jax import lax
from jax.experimental import pallas as pl
from jax.experimental.pallas import tpu as pltpu
from jax.experimental.pallas import tpu_sc as plsc

ORDER = 16
PERIOD = 365.25
TWO_PI = np.float32(2.0 * np.pi)
INV_TWO_PI = np.float32(1.0 / (2.0 * np.pi))
OMEGA = np.float32(2.0 * np.pi / PERIOD)

SIN_C = tuple(
    np.float32(c) for c in
    (0.9999998807907104, -0.16666607558727264, 0.008332732133567333,
     -0.00019816691929008812, 2.7083260647486895e-06,
     -2.069596938270024e-08))
COS_C = tuple(
    np.float32(c) for c in
    (1.0, -0.49999985098838806, 0.041666463017463684,
     -0.0013887732056900859, 2.4769053197815083e-05,
     -2.707544979330123e-07, 1.7243751981865785e-09))

NC = 2
NS = 16
NW = NC * NS
LANES = 16

CHUNK = 1024
N_FULL = 976
TAIL0 = N_FULL * CHUNK
TAILP = 640
CAP = 1024


def _poly_even(c, r2):
    acc = jnp.full((LANES,), c[-1], jnp.float32)
    for coef in reversed(c[:-1]):
        acc = acc * r2 + coef
    return acc


@jax.jit
def _sc_seasonality(ea_t, eb_t, tail_a, tail_b, flat_idx, flat_t):
    B = flat_idx.shape[0]
    mesh = plsc.VectorSubcoreMesh(core_axis_name="c", subcore_axis_name="s")

    @functools.partial(
        pl.kernel,
        out_type=jax.ShapeDtypeStruct((B + 8, 128), jnp.float32),
        mesh=mesh,
        scratch_types=[
            pltpu.VMEM((B,), jnp.int32),
            pltpu.VMEM((B,), jnp.float32),
            pltpu.VMEM((2, ORDER, CHUNK), jnp.float32),
            pltpu.VMEM((2, ORDER, TAILP), jnp.float32),
            pltpu.VMEM((CAP + LANES,), jnp.float32),
            pltpu.VMEM((CAP + LANES,), jnp.int32),
            pltpu.VMEM((LANES, 128), jnp.float32),
            pltpu.SemaphoreType.DMA,
            pltpu.SemaphoreType.DMA,
        ],
        compiler_params=pltpu.CompilerParams(needs_layout_passes=False),
    )
    def season_kernel(ea_hbm, eb_hbm, ta_hbm, tb_hbm, idx_hbm, t_hbm, o_hbm,
                      idx_v, t_v, buf_v, tl_v, vals_v, pos_v, stage_v,
                      sem, sem2):
        wid = lax.axis_index("s") * NC + lax.axis_index("c")
        base_chunks = 30
        n_my = base_chunks + jnp.where(wid < 16, 1, 0)
        start = wid * base_chunks + jnp.minimum(wid, 16)

        pltpu.sync_copy(idx_hbm, idx_v)
        pltpu.sync_copy(t_hbm, t_v)
        pltpu.sync_copy(ta_hbm, tl_v.at[0])
        pltpu.sync_copy(tb_hbm, tl_v.at[1])

        lane = lax.iota(jnp.int32, LANES)
        nvec = (lane + 1).astype(jnp.float32)

        def fourier(a_n, b_n, theta):
            x = theta * nvec
            y = x * INV_TWO_PI
            half = jnp.where(y >= 0.0, 0.5, -0.5).astype(jnp.float32)
            k = (y + half).astype(jnp.int32).astype(jnp.float32)
            r = x - k * TWO_PI
            r2 = r * r
            s = r * _poly_even(SIN_C, r2)
            c = _poly_even(COS_C, r2)
            return jnp.sum(c * a_n + s * b_n)

        def process_item(pos, cur, blk_a, blk_b, col0):
            psp = jnp.full((LANES,), pos, jnp.int32)
            bsp = plsc.load_gather(idx_v, [psp])
            lloc = bsp - col0
            a_n = plsc.load_gather(blk_a, [lane, lloc])
            b_n = plsc.load_gather(blk_b, [lane, lloc])
            theta = plsc.load_gather(t_v, [psp]) * OMEGA
            contrib = fourier(a_n, b_n, theta)
            m1 = lane == 0
            plsc.store_compressed(
                vals_v.at[pl.ds(cur, LANES)],
                jnp.full((LANES,), contrib, jnp.float32), mask=m1)
            plsc.store_compressed(
                pos_v.at[pl.ds(cur, LANES)], psp, mask=m1)
            return cur + 1

        def make_scan(hit_of, blk_a, blk_b, col0):
            def scan(j, cur):
                iv = idx_v[pl.ds(j * LANES, LANES)]
                hit = hit_of(iv)
                nhit = plsc.all_reduce_population_count(hit)[0]

                def do_hits(c0):
                    c = c0
                    for u in range(LANES):
                        msel = hit & (lane == u)
                        has = plsc.all_reduce_population_count(msel)[0]
                        c = lax.cond(
                            (has > 0) & (c < CAP),
                            lambda cc, _u=u: process_item(
                                j * LANES + _u, cc, blk_a, blk_b, col0),
                            lambda cc: cc,
                            c)
                    return c

                return lax.cond(nhit > 0, do_hits, lambda c0: c0, cur)

            return scan

        def chunk_loop(i, cur):
            cid = start + i
            col0 = pl.multiple_of(cid * CHUNK, 128)
            cps = []
            for tbl, s in ((ea_hbm, 0), (eb_hbm, 1)):
                cps.append(pltpu.async_copy(
                    tbl.at[pl.ds(0, 8), pl.ds(col0, CHUNK)],
                    buf_v.at[s, pl.ds(0, 8)], sem))
                cps.append(pltpu.async_copy(
                    tbl.at[pl.ds(8, 8), pl.ds(col0, CHUNK)],
                    buf_v.at[s, pl.ds(8, 8)], sem))
            for c in cps:
                c.wait()
            scan = make_scan(
                lambda iv: lax.shift_right_logical(iv, 10) == cid,
                buf_v.at[0], buf_v.at[1], col0)
            return lax.fori_loop(0, B // LANES, scan, cur)

        cursor = lax.fori_loop(0, n_my, chunk_loop, 0)

        def tail_pass(cur):
            scan = make_scan(
                lambda iv: iv >= TAIL0,
                tl_v.at[0], tl_v.at[1], jnp.int32(TAIL0))
            return lax.fori_loop(0, B // LANES, scan, cur)

        cursor = lax.cond(wid == NW - 1, tail_pass, lambda c: c, cursor)

        def flush(g, carry):
            def do(c):
                pv = pos_v[pl.ds(g * LANES, LANES)]
                okm = (g * LANES + lane) < jnp.full((LANES,), cursor,
                                                    jnp.int32)
                pv = jnp.where(okm, pv, B)
                vv = vals_v[pl.ds(g * LANES, LANES)]
                plsc.store_scatter(
                    stage_v, [lane, jnp.zeros((LANES,), jnp.int32)], vv)
                pltpu.async_copy(stage_v, o_hbm.at[pv], sem2).wait()
                return c

            return lax.cond(g * LANES < cursor, do, lambda c: c, carry)

        lax.fori_loop(0, CAP // LANES, flush, 0)

    return season_kernel(ea_t, eb_t, tail_a, tail_b, flat_idx, flat_t)


def kernel(t, idx, emb_a, emb_b):
    B = idx.shape[0]
    ea_t = emb_a.T
    eb_t = emb_b.T
    pad = TAILP - (emb_a.shape[0] - TAIL0)
    tail_a = jnp.pad(ea_t[:, TAIL0:], ((0, 0), (0, pad)))
    tail_b = jnp.pad(eb_t[:, TAIL0:], ((0, 0), (0, pad)))
    out2d = _sc_seasonality(ea_t, eb_t, tail_a, tail_b,
                            idx.reshape(B), t.reshape(B))
    return out2d[:B, :1]


# speedup vs baseline: 3.1347x; 3.1347x over previous
import functools

import jax
import jax.numpy as jnp
import numpy as np
from jax import lax
from jax.experimental import pallas as pl
from jax.experimental.pallas import tpu as pltpu
from jax.experimental.pallas import tpu_sc as plsc

ORDER = 16
PERIOD = 365.25
TWO_PI = np.float32(2.0 * np.pi)
INV_TWO_PI = np.float32(1.0 / (2.0 * np.pi))
OMEGA = np.float32(2.0 * np.pi / PERIOD)

SIN_C = tuple(
    np.float32(c) for c in
    (0.9999998807907104, -0.16666607558727264, 0.008332732133567333,
     -0.00019816691929008812, 2.7083260647486895e-06,
     -2.069596938270024e-08))
COS_C = tuple(
    np.float32(c) for c in
    (1.0, -0.49999985098838806, 0.041666463017463684,
     -0.0013887732056900859, 2.4769053197815083e-05,
     -2.707544979330123e-07, 1.7243751981865785e-09))

NC = 2
NS = 16
NW = NC * NS
LANES = 16

CHUNK = 1024
N_FULL = 976
TAIL0 = N_FULL * CHUNK
TAILP = 640
CAP = 1024


def _poly_even(c, r2):
    acc = jnp.full((LANES,), c[-1], jnp.float32)
    for coef in reversed(c[:-1]):
        acc = acc * r2 + coef
    return acc


@jax.jit
def _sc_seasonality(ea_t, eb_t, tail_a, tail_b, flat_idx, flat_t):
    B = flat_idx.shape[0]
    mesh = plsc.VectorSubcoreMesh(core_axis_name="c", subcore_axis_name="s")

    @functools.partial(
        pl.kernel,
        out_type=jax.ShapeDtypeStruct((B + 8, 128), jnp.float32),
        mesh=mesh,
        scratch_types=[
            pltpu.VMEM((B,), jnp.int32),
            pltpu.VMEM((B,), jnp.float32),
            pltpu.VMEM((2, ORDER, CHUNK), jnp.float32),
            pltpu.VMEM((2, ORDER, TAILP), jnp.float32),
            pltpu.VMEM((CAP + LANES,), jnp.float32),
            pltpu.VMEM((CAP + LANES,), jnp.int32),
            pltpu.VMEM((CAP + LANES,), jnp.int32),
            pltpu.VMEM((LANES, 128), jnp.float32),
            pltpu.SemaphoreType.DMA,
            pltpu.SemaphoreType.DMA,
        ],
        compiler_params=pltpu.CompilerParams(needs_layout_passes=False),
    )
    def season_kernel(ea_hbm, eb_hbm, ta_hbm, tb_hbm, idx_hbm, t_hbm, o_hbm,
                      idx_v, t_v, buf_v, tl_v, vals_v, pos_v, myhits_v,
                      stage_v, sem, sem2):
        wid = lax.axis_index("s") * NC + lax.axis_index("c")
        base_chunks = 30
        n_my = base_chunks + jnp.where(wid < 16, 1, 0)
        start = wid * base_chunks + jnp.minimum(wid, 16)

        pltpu.sync_copy(idx_hbm, idx_v)
        pltpu.sync_copy(t_hbm, t_v)
        pltpu.sync_copy(ta_hbm, tl_v.at[0])
        pltpu.sync_copy(tb_hbm, tl_v.at[1])

        lane = lax.iota(jnp.int32, LANES)
        nvec = (lane + 1).astype(jnp.float32)

        def fourier(a_n, b_n, theta):
            x = theta * nvec
            y = x * INV_TWO_PI
            half = jnp.where(y >= 0.0, 0.5, -0.5).astype(jnp.float32)
            k = (y + half).astype(jnp.int32).astype(jnp.float32)
            r = x - k * TWO_PI
            r2 = r * r
            s = r * _poly_even(SIN_C, r2)
            c = _poly_even(COS_C, r2)
            return jnp.sum(c * a_n + s * b_n)

        def process_item(g, u, cur, blk_a, blk_b, col0):
            psp = plsc.load_gather(
                myhits_v, [jnp.full((LANES,), g * LANES + u, jnp.int32)])
            bsp = plsc.load_gather(idx_v, [psp])
            lloc = bsp - col0
            a_n = plsc.load_gather(blk_a, [lane, lloc])
            b_n = plsc.load_gather(blk_b, [lane, lloc])
            theta = plsc.load_gather(t_v, [psp]) * OMEGA
            contrib = fourier(a_n, b_n, theta)
            m1 = lane == 0
            plsc.store_compressed(
                vals_v.at[pl.ds(cur, LANES)],
                jnp.full((LANES,), contrib, jnp.float32), mask=m1)
            plsc.store_compressed(
                pos_v.at[pl.ds(cur, LANES)], psp, mask=m1)
            return cur + 1

        is_last = wid == NW - 1

        def precollect(j, cur):
            iv = idx_v[pl.ds(j * LANES, LANES)]
            cid = lax.shift_right_logical(iv, 10)
            mine = (cid >= start) & (cid < start + n_my) & (iv < TAIL0)
            mine = mine | (is_last & (iv >= TAIL0))
            nhit = plsc.all_reduce_population_count(mine)[0]

            def app(c):
                plsc.store_compressed(
                    myhits_v.at[pl.ds(c, LANES)], j * LANES + lane,
                    mask=mine)
                return c + nhit

            return lax.cond((nhit > 0) & (cur + LANES <= CAP), app,
                            lambda c: c, cur)

        count = lax.fori_loop(0, B // LANES, precollect, 0)
        n_groups = lax.shift_right_logical(count + LANES - 1, 4)

        def make_scan(hit_of, blk_a, blk_b, col0):
            def scan(g, cur):
                pv = myhits_v[pl.ds(g * LANES, LANES)]
                valid = (g * LANES + lane) < jnp.full((LANES,), count,
                                                      jnp.int32)
                pv = jnp.where(valid, pv, 0)
                iv = plsc.load_gather(idx_v, [pv])
                hit = hit_of(iv) & valid
                nhit = plsc.all_reduce_population_count(hit)[0]

                def do_hits(c0):
                    c = c0
                    for u in range(LANES):
                        msel = hit & (lane == u)
                        has = plsc.all_reduce_population_count(msel)[0]
                        c = lax.cond(
                            (has > 0) & (c < CAP),
                            lambda cc, _u=u: process_item(
                                g, _u, cc, blk_a, blk_b, col0),
                            lambda cc: cc,
                            c)
                    return c

                return lax.cond(nhit > 0, do_hits, lambda c0: c0, cur)

            return scan

        def chunk_loop(i, cur):
            cid = start + i
            col0 = pl.multiple_of(cid * CHUNK, 128)
            cps = []
            for tbl, s in ((ea_hbm, 0), (eb_hbm, 1)):
                cps.append(pltpu.async_copy(
                    tbl.at[pl.ds(0, 8), pl.ds(col0, CHUNK)],
                    buf_v.at[s, pl.ds(0, 8)], sem))
                cps.append(pltpu.async_copy(
                    tbl.at[pl.ds(8, 8), pl.ds(col0, CHUNK)],
                    buf_v.at[s, pl.ds(8, 8)], sem))
            for c in cps:
                c.wait()
            scan = make_scan(
                lambda iv: lax.shift_right_logical(iv, 10) == cid,
                buf_v.at[0], buf_v.at[1], col0)
            return lax.fori_loop(0, n_groups, scan, cur)

        cursor = lax.fori_loop(0, n_my, chunk_loop, 0)

        def tail_pass(cur):
            scan = make_scan(
                lambda iv: iv >= TAIL0,
                tl_v.at[0], tl_v.at[1], jnp.int32(TAIL0))
            return lax.fori_loop(0, n_groups, scan, cur)

        cursor = lax.cond(is_last, tail_pass, lambda c: c, cursor)

        def flush(g, carry):
            def do(c):
                pv = pos_v[pl.ds(g * LANES, LANES)]
                okm = (g * LANES + lane) < jnp.full((LANES,), cursor,
                                                    jnp.int32)
                pv = jnp.where(okm, pv, B)
                vv = vals_v[pl.ds(g * LANES, LANES)]
                plsc.store_scatter(
                    stage_v, [lane, jnp.zeros((LANES,), jnp.int32)], vv)
                pltpu.async_copy(stage_v, o_hbm.at[pv], sem2).wait()
                return c

            return lax.cond(g * LANES < cursor, do, lambda c: c, carry)

        lax.fori_loop(0, CAP // LANES, flush, 0)

    return season_kernel(ea_t, eb_t, tail_a, tail_b, flat_idx, flat_t)


def kernel(t, idx, emb_a, emb_b):
    B = idx.shape[0]
    ea_t = emb_a.T
    eb_t = emb_b.T
    pad = TAILP - (emb_a.shape[0] - TAIL0)
    tail_a = jnp.pad(ea_t[:, TAIL0:], ((0, 0), (0, pad)))
    tail_b = jnp.pad(eb_t[:, TAIL0:], ((0, 0), (0, pad)))
    out2d = _sc_seasonality(ea_t, eb_t, tail_a, tail_b,
                            idx.reshape(B), t.reshape(B))
    return out2d[:B, :1]


# speedup vs baseline: 4.3756x; 1.3958x over previous
import functools

import jax
import jax.numpy as jnp
import numpy as np
from jax import lax
from jax.experimental import pallas as pl
from jax.experimental.pallas import tpu as pltpu
from jax.experimental.pallas import tpu_sc as plsc

ORDER = 16
PERIOD = 365.25
TWO_PI = np.float32(2.0 * np.pi)
INV_TWO_PI = np.float32(1.0 / (2.0 * np.pi))
OMEGA = np.float32(2.0 * np.pi / PERIOD)

SIN_C = tuple(
    np.float32(c) for c in
    (0.9999998807907104, -0.16666607558727264, 0.008332732133567333,
     -0.00019816691929008812, 2.7083260647486895e-06,
     -2.069596938270024e-08))
COS_C = tuple(
    np.float32(c) for c in
    (1.0, -0.49999985098838806, 0.041666463017463684,
     -0.0013887732056900859, 2.4769053197815083e-05,
     -2.707544979330123e-07, 1.7243751981865785e-09))

NC = 2
NS = 16
NW = NC * NS
LANES = 16

CHUNK = 2048
N_FULL = 488
TAIL0 = N_FULL * CHUNK
TAILP = 640
CAP = 1024


def _poly_even(c, r2):
    acc = jnp.full((LANES,), c[-1], jnp.float32)
    for coef in reversed(c[:-1]):
        acc = acc * r2 + coef
    return acc


@jax.jit
def _sc_seasonality(ea_t, eb_t, tail_a, tail_b, flat_idx, flat_t):
    B = flat_idx.shape[0]
    mesh = plsc.VectorSubcoreMesh(core_axis_name="c", subcore_axis_name="s")

    @functools.partial(
        pl.kernel,
        out_type=jax.ShapeDtypeStruct((B + 8, 128), jnp.float32),
        mesh=mesh,
        scratch_types=[
            pltpu.VMEM((B,), jnp.int32),
            pltpu.VMEM((B,), jnp.float32),
            pltpu.VMEM((2, ORDER, CHUNK), jnp.float32),
            pltpu.VMEM((2, ORDER, TAILP), jnp.float32),
            pltpu.VMEM((CAP + LANES,), jnp.float32),
            pltpu.VMEM((CAP + LANES,), jnp.int32),
            pltpu.VMEM((CAP + LANES,), jnp.int32),
            pltpu.VMEM((LANES, 128), jnp.float32),
            pltpu.SemaphoreType.DMA,
            pltpu.SemaphoreType.DMA,
        ],
        compiler_params=pltpu.CompilerParams(needs_layout_passes=False),
    )
    def season_kernel(ea_hbm, eb_hbm, ta_hbm, tb_hbm, idx_hbm, t_hbm, o_hbm,
                      idx_v, t_v, buf_v, tl_v, vals_v, pos_v, myhits_v,
                      stage_v, sem, sem2):
        wid = lax.axis_index("s") * NC + lax.axis_index("c")
        base_chunks = 15
        n_my = base_chunks + jnp.where(wid < 8, 1, 0)
        start = wid * base_chunks + jnp.minimum(wid, 8)

        pltpu.sync_copy(idx_hbm, idx_v)
        pltpu.sync_copy(t_hbm, t_v)
        pltpu.sync_copy(ta_hbm, tl_v.at[0])
        pltpu.sync_copy(tb_hbm, tl_v.at[1])

        lane = lax.iota(jnp.int32, LANES)
        nvec = (lane + 1).astype(jnp.float32)

        def process_group(pv, iv, hit, nhit, cur, blk_a, blk_b, col0):
            lloc = jnp.where(hit, iv - col0, 0)
            theta = plsc.load_gather(t_v, [pv]) * OMEGA
            y = theta * INV_TWO_PI
            half = jnp.where(y >= 0.0, 0.5, -0.5).astype(jnp.float32)
            k = (y + half).astype(jnp.int32).astype(jnp.float32)
            r = theta - k * TWO_PI
            r2 = r * r
            s1 = r * _poly_even(SIN_C, r2)
            c1 = _poly_even(COS_C, r2)
            two_c1 = c1 + c1
            def harmonic(n, carry):
                acc, c_prev, c_cur, s_prev, s_cur = carry
                col = jnp.full((LANES,), n, jnp.int32)
                a_n = plsc.load_gather(blk_a, [col, lloc])
                b_n = plsc.load_gather(blk_b, [col, lloc])
                acc = acc + c_cur * a_n + s_cur * b_n
                c_next = two_c1 * c_cur - c_prev
                s_next = two_c1 * s_cur - s_prev
                return acc, c_cur, c_next, s_cur, s_next

            acc, _, _, _, _ = lax.fori_loop(
                0, ORDER, harmonic,
                (jnp.zeros((LANES,), jnp.float32),
                 jnp.ones((LANES,), jnp.float32), c1,
                 jnp.zeros((LANES,), jnp.float32), s1))
            plsc.store_compressed(
                vals_v.at[pl.ds(cur, LANES)], acc, mask=hit)
            plsc.store_compressed(
                pos_v.at[pl.ds(cur, LANES)], pv, mask=hit)
            return cur + nhit

        is_last = wid == NW - 1

        def precollect(j, cur):
            iv = idx_v[pl.ds(j * LANES, LANES)]
            cid = lax.shift_right_logical(iv, 11)
            mine = (cid >= start) & (cid < start + n_my) & (iv < TAIL0)
            mine = mine | (is_last & (iv >= TAIL0))
            nhit = plsc.all_reduce_population_count(mine)[0]

            def app(c):
                plsc.store_compressed(
                    myhits_v.at[pl.ds(c, LANES)], j * LANES + lane,
                    mask=mine)
                return c + nhit

            return lax.cond((nhit > 0) & (cur + LANES <= CAP), app,
                            lambda c: c, cur)

        count = lax.fori_loop(0, B // LANES, precollect, 0)
        n_groups = lax.shift_right_logical(count + LANES - 1, 4)

        def make_scan(hit_of, blk_a, blk_b, col0):
            def scan(g, cur):
                pv = myhits_v[pl.ds(g * LANES, LANES)]
                valid = (g * LANES + lane) < jnp.full((LANES,), count,
                                                      jnp.int32)
                pv = jnp.where(valid, pv, 0)
                iv = plsc.load_gather(idx_v, [pv])
                hit = hit_of(iv) & valid
                nhit = plsc.all_reduce_population_count(hit)[0]

                return lax.cond(
                    (nhit > 0) & (cur + LANES <= CAP),
                    lambda c0: process_group(pv, iv, hit, nhit, c0,
                                             blk_a, blk_b, col0),
                    lambda c0: c0,
                    cur)

            return scan

        def chunk_loop(i, cur):
            cid = start + i
            col0 = pl.multiple_of(cid * CHUNK, 128)
            cps = []
            for tbl, s in ((ea_hbm, 0), (eb_hbm, 1)):
                cps.append(pltpu.async_copy(
                    tbl.at[pl.ds(0, 8), pl.ds(col0, CHUNK)],
                    buf_v.at[s, pl.ds(0, 8)], sem))
                cps.append(pltpu.async_copy(
                    tbl.at[pl.ds(8, 8), pl.ds(col0, CHUNK)],
                    buf_v.at[s, pl.ds(8, 8)], sem))
            for c in cps:
                c.wait()
            scan = make_scan(
                lambda iv: lax.shift_right_logical(iv, 11) == cid,
                buf_v.at[0], buf_v.at[1], col0)
            return lax.fori_loop(0, n_groups, scan, cur)

        cursor = lax.fori_loop(0, n_my, chunk_loop, 0)

        def tail_pass(cur):
            scan = make_scan(
                lambda iv: iv >= TAIL0,
                tl_v.at[0], tl_v.at[1], jnp.int32(TAIL0))
            return lax.fori_loop(0, n_groups, scan, cur)

        cursor = lax.cond(is_last, tail_pass, lambda c: c, cursor)

        def flush(g, carry):
            def do(c):
                pv = pos_v[pl.ds(g * LANES, LANES)]
                okm = (g * LANES + lane) < jnp.full((LANES,), cursor,
                                                    jnp.int32)
                pv = jnp.where(okm, pv, B)
                vv = vals_v[pl.ds(g * LANES, LANES)]
                plsc.store_scatter(
                    stage_v, [lane, jnp.zeros((LANES,), jnp.int32)], vv)
                pltpu.async_copy(stage_v, o_hbm.at[pv], sem2).wait()
                return c

            return lax.cond(g * LANES < cursor, do, lambda c: c, carry)

        lax.fori_loop(0, CAP // LANES, flush, 0)

    return season_kernel(ea_t, eb_t, tail_a, tail_b, flat_idx, flat_t)


def kernel(t, idx, emb_a, emb_b):
    B = idx.shape[0]
    ea_t = emb_a.T
    eb_t = emb_b.T
    pad = TAILP - (emb_a.shape[0] - TAIL0)
    tail_a = jnp.pad(ea_t[:, TAIL0:], ((0, 0), (0, pad)))
    tail_b = jnp.pad(eb_t[:, TAIL0:], ((0, 0), (0, pad)))
    out2d = _sc_seasonality(ea_t, eb_t, tail_a, tail_b,
                            idx.reshape(B), t.reshape(B))
    return out2d[:B, :1]
